# baseline (speedup 1.0000x reference)
import functools

import jax
import jax.numpy as jnp
from jax import lax
from jax.experimental import pallas as pl
from jax.experimental.pallas import tpu as pltpu
from jax.experimental.pallas import tpu_sc as plsc

_N0 = 2500.0
_LANES = 16


def _preprocess_sc(x, patt, *, num_cores, num_subcores, interpret=False):
    num_workers = num_cores * num_subcores
    bs, cs, two_m = x.shape
    m = patt.shape[0]
    rows_per_w = bs * cs // num_workers
    vecs_per_row = m // _LANES

    mesh = plsc.VectorSubcoreMesh(
        core_axis_name="c", subcore_axis_name="s",
        num_cores=num_cores, num_subcores=num_subcores,
    )

    @functools.partial(
        pl.kernel,
        out_type=jax.ShapeDtypeStruct((bs * cs, m), jnp.float32),
        mesh=mesh,
        scratch_types=[
            pltpu.VMEM((rows_per_w, cs, two_m), jnp.float32),
            pltpu.VMEM((m,), jnp.float32),
            pltpu.VMEM((rows_per_w, m), jnp.float32),
        ],
        compiler_params=pltpu.CompilerParams(needs_layout_passes=False),
        interpret=interpret,
    )
    def run(x_hbm, patt_hbm, out_hbm, x_v, patt_v, out_v):
        wid = lax.axis_index("s") * num_cores + lax.axis_index("c")
        pltpu.sync_copy(x_hbm.at[pl.ds(wid * rows_per_w, rows_per_w)], x_v)
        pltpu.sync_copy(patt_hbm, patt_v)

        even_iota = 2 * lax.iota(jnp.int32, _LANES)
        odd_iota = even_iota + 1
        scale = jnp.float32(2.0 / _N0)
        zero = jnp.zeros((_LANES,), jnp.int32)

        for r in range(rows_per_w):
            @plsc.parallel_loop(0, vecs_per_row, 1, unroll=8)
            def body(j, r=r):
                base = j * 32
                rr = zero + r
                even = plsc.load_gather(x_v, [rr, zero, base + even_iota])
                odd = plsc.load_gather(x_v, [rr, zero, base + odd_iota])
                p = patt_v[pl.ds(j * _LANES, _LANES)]
                out_v[r, pl.ds(j * _LANES, _LANES)] = (even - odd) * scale - p

        pltpu.sync_copy(out_v, out_hbm.at[pl.ds(wid * rows_per_w, rows_per_w)])

    return run(x, patt)


def kernel(x, Patt, b, c, h, w):
    bs, cs, two_m = x.shape
    m = Patt.shape[0]
    info = plsc.get_sparse_core_info()
    out = _preprocess_sc(x, Patt.astype(jnp.float32),
                         num_cores=info.num_cores,
                         num_subcores=info.num_subcores)
    return jnp.reshape(out, (bs, cs, m))


# speedup vs baseline: 1.2218x; 1.2218x over previous
import functools

import jax
import jax.numpy as jnp
from jax import lax
from jax.experimental import pallas as pl
from jax.experimental.pallas import tpu as pltpu
from jax.experimental.pallas import tpu_sc as plsc

_N0 = 2500.0
_LANES = 16


def _preprocess_sc(x2, patt, *, num_cores, num_subcores, interpret=False):
    num_workers = num_cores * num_subcores
    in_rows, lanes = x2.shape
    m = patt.shape[0]
    out_rows = in_rows // 2
    irows_per_w = in_rows // num_workers
    orows_per_w = out_rows // num_workers
    vecs_per_w = orows_per_w * lanes // _LANES
    vecs_per_orow = lanes // _LANES
    vecs_per_mrow = m // _LANES

    mesh = plsc.VectorSubcoreMesh(
        core_axis_name="c", subcore_axis_name="s",
        num_cores=num_cores, num_subcores=num_subcores,
    )

    @functools.partial(
        pl.kernel,
        out_type=jax.ShapeDtypeStruct((out_rows, lanes), jnp.float32),
        mesh=mesh,
        scratch_types=[
            pltpu.VMEM((irows_per_w, lanes), jnp.float32),
            pltpu.VMEM((m,), jnp.float32),
            pltpu.VMEM((orows_per_w, lanes), jnp.float32),
        ],
        compiler_params=pltpu.CompilerParams(needs_layout_passes=False),
        interpret=interpret,
    )
    def run(x_hbm, patt_hbm, out_hbm, x_v, patt_v, out_v):
        wid = lax.axis_index("s") * num_cores + lax.axis_index("c")
        pltpu.sync_copy(x_hbm.at[pl.ds(wid * irows_per_w, irows_per_w)], x_v)
        pltpu.sync_copy(patt_hbm, patt_v)

        even_iota = 2 * lax.iota(jnp.int32, _LANES)
        odd_iota = even_iota + 1
        zero = jnp.zeros((_LANES,), jnp.int32)
        scale = jnp.float32(2.0 / _N0)

        @plsc.parallel_loop(0, vecs_per_w, 1, unroll=8)
        def body(v):
            irow = zero + lax.shift_right_logical(v, 2)
            cbase = lax.shift_left(lax.bitwise_and(v, 3), 5)
            even = plsc.load_gather(x_v, [irow, cbase + even_iota])
            odd = plsc.load_gather(x_v, [irow, cbase + odd_iota])
            pm = lax.bitwise_and(v, vecs_per_mrow - 1) * _LANES
            p = patt_v[pl.ds(pm, _LANES)]
            orow = lax.shift_right_logical(v, 3)
            ocol = lax.shift_left(lax.bitwise_and(v, 7), 4)
            out_v[orow, pl.ds(ocol, _LANES)] = (even - odd) * scale - p

        pltpu.sync_copy(out_v, out_hbm.at[pl.ds(wid * orows_per_w, orows_per_w)])

    return run(x2, patt)


def kernel(x, Patt, b, c, h, w):
    bs, cs, two_m = x.shape
    m = Patt.shape[0]
    lanes = 128
    x2 = jnp.reshape(x, (bs * cs * two_m // lanes, lanes))
    info = plsc.get_sparse_core_info()
    out = _preprocess_sc(x2, Patt.astype(jnp.float32),
                         num_cores=info.num_cores,
                         num_subcores=info.num_subcores)
    return jnp.reshape(out, (bs, cs, m))


# speedup vs baseline: 1.2231x; 1.0011x over previous
import functools

import jax
import jax.numpy as jnp
from jax import lax
from jax.experimental import pallas as pl
from jax.experimental.pallas import tpu as pltpu
from jax.experimental.pallas import tpu_sc as plsc

_N0 = 2500.0
_LANES = 16


def _preprocess_sc(x2, patt, *, num_cores, num_subcores, interpret=False):
    num_workers = num_cores * num_subcores
    in_rows, lanes = x2.shape
    m = patt.shape[0]
    out_rows = in_rows // 2
    irows_per_w = in_rows // num_workers
    orows_per_w = out_rows // num_workers
    vecs_per_w = orows_per_w * lanes // _LANES
    vecs_per_orow = lanes // _LANES
    vecs_per_mrow = m // _LANES

    mesh = plsc.VectorSubcoreMesh(
        core_axis_name="c", subcore_axis_name="s",
        num_cores=num_cores, num_subcores=num_subcores,
    )

    @functools.partial(
        pl.kernel,
        out_type=jax.ShapeDtypeStruct((out_rows, lanes), jnp.float32),
        mesh=mesh,
        scratch_types=[
            pltpu.VMEM((irows_per_w, lanes), jnp.float32),
            pltpu.VMEM((m,), jnp.float32),
            pltpu.VMEM((orows_per_w, lanes), jnp.float32),
        ],
        compiler_params=pltpu.CompilerParams(
            needs_layout_passes=False,
            use_tc_tiling_on_sc=True,
        ),
        interpret=interpret,
    )
    def run(x_hbm, patt_hbm, out_hbm, x_v, patt_v, out_v):
        wid = lax.axis_index("s") * num_cores + lax.axis_index("c")
        pltpu.sync_copy(x_hbm.at[pl.ds(wid * irows_per_w, irows_per_w)], x_v)
        pltpu.sync_copy(patt_hbm, patt_v)

        even_iota = 2 * lax.iota(jnp.int32, _LANES)
        odd_iota = even_iota + 1
        zero = jnp.zeros((_LANES,), jnp.int32)
        scale = jnp.float32(2.0 / _N0)

        @plsc.parallel_loop(0, vecs_per_w, 1, unroll=8)
        def body(v):
            irow = zero + lax.shift_right_logical(v, 2)
            cbase = lax.shift_left(lax.bitwise_and(v, 3), 5)
            even = plsc.load_gather(x_v, [irow, cbase + even_iota])
            odd = plsc.load_gather(x_v, [irow, cbase + odd_iota])
            pm = lax.bitwise_and(v, vecs_per_mrow - 1) * _LANES
            p = patt_v[pl.ds(pm, _LANES)]
            orow = lax.shift_right_logical(v, 3)
            ocol = lax.shift_left(lax.bitwise_and(v, 7), 4)
            out_v[orow, pl.ds(ocol, _LANES)] = (even - odd) * scale - p

        pltpu.sync_copy(out_v, out_hbm.at[pl.ds(wid * orows_per_w, orows_per_w)])

    return run(x2, patt)


def kernel(x, Patt, b, c, h, w):
    bs, cs, two_m = x.shape
    m = Patt.shape[0]
    lanes = 128
    x2 = jnp.reshape(x, (bs * cs * two_m // lanes, lanes))
    info = plsc.get_sparse_core_info()
    out = _preprocess_sc(x2, Patt.astype(jnp.float32),
                         num_cores=info.num_cores,
                         num_subcores=info.num_subcores)
    return jnp.reshape(out, (bs, cs, m))
